# baseline (device time: 130861 ns/iter reference)
import jax
import jax.numpy as jnp
from jax import lax
from jax.experimental import pallas as pl
from jax.experimental.pallas import tpu as pltpu

N_DEV = 16
NP = 4
B, S, D = 2, 512, 2048
DC = 128
H, DH, DR = 16, 128, 32
T = B * S
TH = T // 2
BW = NP * DH
SCALE = (DH + DR) ** -0.5


def _neighbors(my):
    z = my // NP
    s = lax.rem(my, NP)
    right = NP * z + lax.rem(s + 1, NP)
    left = NP * z + lax.rem(s + NP - 1, NP)
    up = NP * lax.rem(z + 1, NP) + s
    down = NP * lax.rem(z + NP - 1, NP) + s
    return z, s, right, left, up, down


def _barrier(nbrs):
    bsem = pltpu.get_barrier_semaphore()
    for nbr in nbrs:
        pl.semaphore_signal(bsem, inc=1, device_id=(nbr,),
                            device_id_type=pl.DeviceIdType.MESH)
    pl.semaphore_wait(bsem, len(nbrs))


def _rs_body(x_ref, wdkv_ref, wukp_ref, wuvp_ref, wq_ref, wqr_ref, wkr_ref,
             kv_ref, q_ref, qr_ref, kr_ref,
             st1k_ref, st1v_ref, r1k_ref, r1v_ref,
             st2k_ref, st2v_ref, r2k_ref, r2v_ref, kb_ref, vb_ref,
             s1k_sems, r1k_sems, s1v_sems, r1v_sems,
             s2k_sems, r2k_sems, s2v_sems, r2v_sems):
    my = lax.axis_index("i")
    z, s, right, left, up, down = _neighbors(my)
    _barrier((right, left, up, down))

    x = x_ref[...]
    c = jnp.dot(x, wdkv_ref[...],
                preferred_element_type=jnp.float32).astype(jnp.bfloat16)

    def bundle_k(sp):
        return jnp.dot(c, wukp_ref[:, pl.ds(sp * BW, BW)],
                       preferred_element_type=jnp.float32)

    def bundle_v(sp):
        return jnp.dot(c, wuvp_ref[:, pl.ds(sp * BW, BW)],
                       preferred_element_type=jnp.float32)

    k1, v1 = [], []
    for t in range(NP - 1):
        p = t % 2
        if t >= 2:
            k1[t - 2].wait_send()
        kacc = bundle_k(lax.rem(s + NP - t - 1, NP))
        if t >= 1:
            k1[t - 1].wait_recv()
            kacc = kacc + r1k_ref[t - 1].astype(jnp.float32)
        st1k_ref[p, :, :] = kacc.astype(jnp.bfloat16)
        rdma = pltpu.make_async_remote_copy(
            src_ref=st1k_ref.at[p], dst_ref=r1k_ref.at[t],
            send_sem=s1k_sems.at[p], recv_sem=r1k_sems.at[t],
            device_id=(right,), device_id_type=pl.DeviceIdType.MESH)
        rdma.start()
        k1.append(rdma)

        if t >= 2:
            v1[t - 2].wait_send()
        vacc = bundle_v(lax.rem(s + t + 1, NP))
        if t >= 1:
            v1[t - 1].wait_recv()
            vacc = vacc + r1v_ref[t - 1].astype(jnp.float32)
        st1v_ref[p, :, :] = vacc.astype(jnp.bfloat16)
        rdma = pltpu.make_async_remote_copy(
            src_ref=st1v_ref.at[p], dst_ref=r1v_ref.at[t],
            send_sem=s1v_sems.at[p], recv_sem=r1v_sems.at[t],
            device_id=(left,), device_id_type=pl.DeviceIdType.MESH)
        rdma.start()
        v1.append(rdma)

        if t == 0:
            q_ref[...] = jnp.dot(
                x, wq_ref[...],
                preferred_element_type=jnp.float32).astype(jnp.bfloat16)
            qr_ref[...] = jnp.dot(
                x, wqr_ref[...],
                preferred_element_type=jnp.float32).astype(jnp.bfloat16)
            kr_ref[...] = jnp.dot(
                x, wkr_ref[...],
                preferred_element_type=jnp.float32).astype(jnp.bfloat16)

    k1[NP - 2].wait_recv()
    v1[NP - 2].wait_recv()
    kb_ref[...] = bundle_k(s) + r1k_ref[NP - 2].astype(jnp.float32)
    vb_ref[...] = bundle_v(s) + r1v_ref[NP - 2].astype(jnp.float32)

    def kb_slot(zp):
        return kb_ref[:, pl.ds(zp * DH, DH)]

    def vb_slot(zp):
        return vb_ref[:, pl.ds(zp * DH, DH)]

    k2, v2 = [], []
    for t in range(NP - 1):
        p = t % 2
        if t >= 2:
            k2[t - 2].wait_send()
        kacc = kb_slot(lax.rem(z + NP - t - 1, NP))
        if t >= 1:
            k2[t - 1].wait_recv()
            kacc = kacc + r2k_ref[t - 1].astype(jnp.float32)
        st2k_ref[p, :, :] = kacc.astype(jnp.bfloat16)
        rdma = pltpu.make_async_remote_copy(
            src_ref=st2k_ref.at[p], dst_ref=r2k_ref.at[t],
            send_sem=s2k_sems.at[p], recv_sem=r2k_sems.at[t],
            device_id=(up,), device_id_type=pl.DeviceIdType.MESH)
        rdma.start()
        k2.append(rdma)

        if t >= 2:
            v2[t - 2].wait_send()
        vacc = vb_slot(lax.rem(z + t + 1, NP))
        if t >= 1:
            v2[t - 1].wait_recv()
            vacc = vacc + r2v_ref[t - 1].astype(jnp.float32)
        st2v_ref[p, :, :] = vacc.astype(jnp.bfloat16)
        rdma = pltpu.make_async_remote_copy(
            src_ref=st2v_ref.at[p], dst_ref=r2v_ref.at[t],
            send_sem=s2v_sems.at[p], recv_sem=r2v_sems.at[t],
            device_id=(down,), device_id_type=pl.DeviceIdType.MESH)
        rdma.start()
        v2.append(rdma)

    k2[NP - 2].wait_recv()
    kv_ref[:, 0:DH] = (kb_slot(z)
                       + r2k_ref[NP - 2].astype(jnp.float32)
                       ).astype(jnp.bfloat16)
    v2[NP - 2].wait_recv()
    kv_ref[:, DH:2 * DH] = (vb_slot(z)
                            + r2v_ref[NP - 2].astype(jnp.float32)
                            ).astype(jnp.bfloat16)

    for rdmas in (k1, v1, k2, v2):
        rdmas[NP - 3].wait_send()
        rdmas[NP - 2].wait_send()


def _attn_ag_body(kv_ref, q_ref, qr_ref, kr_ref, wop_ref, out_ref,
                  o_ref, bt_ref, bb_ref, zat_ref, zab_ref, rbt_ref, rbb_ref,
                  zat_ssems, zat_rsems, zab_ssems, zab_rsems,
                  bt_ssems, bt_rsems, bb_ssems, bb_rsems):
    my = lax.axis_index("i")
    z, s, right, left, up, down = _neighbors(my)
    _barrier((right, left, up, down))

    def attention(b):
        sl = slice(b * S, (b + 1) * S)
        kh = kv_ref[sl, 0:DH]
        vh = kv_ref[sl, DH:2 * DH]
        s1 = lax.dot_general(q_ref[sl, :], kh, (((1,), (1,)), ((), ())),
                             preferred_element_type=jnp.float32)
        s2 = lax.dot_general(qr_ref[sl, :], kr_ref[sl, :],
                             (((1,), (1,)), ((), ())),
                             preferred_element_type=jnp.float32)
        sc = (s1 + s2) * SCALE
        mx = jnp.max(sc, axis=1, keepdims=True)
        e = jnp.exp(sc - mx)
        pmat = (e / jnp.sum(e, axis=1, keepdims=True)).astype(jnp.bfloat16)
        ob = jnp.dot(pmat, vh, preferred_element_type=jnp.float32)
        o_ref[pl.ds(b * S, S), :] = ob.astype(jnp.bfloat16)

    at, ab = [], []
    attention(0)
    rdma = pltpu.make_async_remote_copy(
        src_ref=o_ref.at[pl.ds(0, TH)], dst_ref=zat_ref.at[0],
        send_sem=zat_ssems.at[0], recv_sem=zat_rsems.at[0],
        device_id=(up,), device_id_type=pl.DeviceIdType.MESH)
    rdma.start()
    at.append(rdma)
    attention(1)
    rdma = pltpu.make_async_remote_copy(
        src_ref=o_ref.at[pl.ds(TH, TH)], dst_ref=zab_ref.at[0],
        send_sem=zab_ssems.at[0], recv_sem=zab_rsems.at[0],
        device_id=(down,), device_id_type=pl.DeviceIdType.MESH)
    rdma.start()
    ab.append(rdma)
    for t in range(1, NP - 1):
        rdma = pltpu.make_async_remote_copy(
            src_ref=zat_ref.at[t - 1], dst_ref=zat_ref.at[t],
            send_sem=zat_ssems.at[t], recv_sem=zat_rsems.at[t],
            device_id=(up,), device_id_type=pl.DeviceIdType.MESH)
        at[t - 1].wait_recv()
        rdma.start()
        at.append(rdma)
        rdma = pltpu.make_async_remote_copy(
            src_ref=zab_ref.at[t - 1], dst_ref=zab_ref.at[t],
            send_sem=zab_ssems.at[t], recv_sem=zab_rsems.at[t],
            device_id=(down,), device_id_type=pl.DeviceIdType.MESH)
        ab[t - 1].wait_recv()
        rdma.start()
        ab.append(rdma)

    bt_ref[:, pl.ds(z * DH, DH)] = o_ref[pl.ds(0, TH), :]
    bb_ref[:, pl.ds(z * DH, DH)] = o_ref[pl.ds(TH, TH), :]
    at[NP - 2].wait_recv()
    ab[NP - 2].wait_recv()
    for t in range(NP - 1):
        zt = lax.rem(z + NP - t - 1, NP)
        bt_ref[:, pl.ds(zt * DH, DH)] = zat_ref[t]
        zbo = lax.rem(z + t + 1, NP)
        bb_ref[:, pl.ds(zbo * DH, DH)] = zab_ref[t]

    gt, gb = [], []
    for t in range(NP - 1):
        rdma = pltpu.make_async_remote_copy(
            src_ref=bt_ref if t == 0 else rbt_ref.at[t - 1],
            dst_ref=rbt_ref.at[t],
            send_sem=bt_ssems.at[t], recv_sem=bt_rsems.at[t],
            device_id=(right,), device_id_type=pl.DeviceIdType.MESH)
        if t >= 1:
            gt[t - 1].wait_recv()
        rdma.start()
        gt.append(rdma)
        rdma = pltpu.make_async_remote_copy(
            src_ref=bb_ref if t == 0 else rbb_ref.at[t - 1],
            dst_ref=rbb_ref.at[t],
            send_sem=bb_ssems.at[t], recv_sem=bb_rsems.at[t],
            device_id=(left,), device_id_type=pl.DeviceIdType.MESH)
        if t >= 1:
            gb[t - 1].wait_recv()
        rdma.start()
        gb.append(rdma)
        if t == 0:
            out_ref[pl.ds(0, TH), :] = jnp.dot(
                bt_ref[...], wop_ref[s],
                preferred_element_type=jnp.float32)
            out_ref[pl.ds(TH, TH), :] = jnp.dot(
                bb_ref[...], wop_ref[s],
                preferred_element_type=jnp.float32)
        else:
            st = lax.rem(s + NP - t, NP)
            sb = lax.rem(s + t, NP)
            out_ref[pl.ds(0, TH), :] += jnp.dot(
                rbt_ref[t - 1], wop_ref[st],
                preferred_element_type=jnp.float32)
            out_ref[pl.ds(TH, TH), :] += jnp.dot(
                rbb_ref[t - 1], wop_ref[sb],
                preferred_element_type=jnp.float32)
    st = lax.rem(s + 1, NP)
    sb = lax.rem(s + NP - 1, NP)
    gt[NP - 2].wait_recv()
    out_ref[pl.ds(0, TH), :] += jnp.dot(
        rbt_ref[NP - 2], wop_ref[st], preferred_element_type=jnp.float32)
    gb[NP - 2].wait_recv()
    out_ref[pl.ds(TH, TH), :] += jnp.dot(
        rbb_ref[NP - 2], wop_ref[sb], preferred_element_type=jnp.float32)

    for rdmas in (at, ab, gt, gb):
        for rdma in rdmas:
            rdma.wait_send()


def kernel(x, Wdkv, Wuk, Wuv, Wq, Wqr, Wkr, Wo):
    my = lax.axis_index("i")
    xb = x.reshape(T, D).astype(jnp.bfloat16)
    wq_h = lax.dynamic_slice(Wq, (0, my * DH), (D, DH)).astype(jnp.bfloat16)
    wqr_h = lax.dynamic_slice(Wqr, (0, my * DR), (D, DR)).astype(jnp.bfloat16)

    def col_perm(w):
        return (w.astype(jnp.bfloat16)
                .reshape(DC, NP, NP, DH)
                .transpose(0, 2, 1, 3)
                .reshape(DC, H * DH))

    kv, q, qr, kr = pl.pallas_call(
        _rs_body,
        out_shape=[
            jax.ShapeDtypeStruct((T, 2 * DH), jnp.bfloat16),
            jax.ShapeDtypeStruct((T, DH), jnp.bfloat16),
            jax.ShapeDtypeStruct((T, DR), jnp.bfloat16),
            jax.ShapeDtypeStruct((T, DR), jnp.bfloat16),
        ],
        in_specs=[pl.BlockSpec(memory_space=pltpu.VMEM)] * 7,
        out_specs=[pl.BlockSpec(memory_space=pltpu.VMEM)] * 4,
        scratch_shapes=[
            pltpu.VMEM((2, T, BW), jnp.bfloat16),
            pltpu.VMEM((2, T, BW), jnp.bfloat16),
            pltpu.VMEM((NP - 1, T, BW), jnp.bfloat16),
            pltpu.VMEM((NP - 1, T, BW), jnp.bfloat16),
            pltpu.VMEM((2, T, DH), jnp.bfloat16),
            pltpu.VMEM((2, T, DH), jnp.bfloat16),
            pltpu.VMEM((NP - 1, T, DH), jnp.bfloat16),
            pltpu.VMEM((NP - 1, T, DH), jnp.bfloat16),
            pltpu.VMEM((T, BW), jnp.float32),
            pltpu.VMEM((T, BW), jnp.float32),
            pltpu.SemaphoreType.DMA((2,)),
            pltpu.SemaphoreType.DMA((NP - 1,)),
            pltpu.SemaphoreType.DMA((2,)),
            pltpu.SemaphoreType.DMA((NP - 1,)),
            pltpu.SemaphoreType.DMA((2,)),
            pltpu.SemaphoreType.DMA((NP - 1,)),
            pltpu.SemaphoreType.DMA((2,)),
            pltpu.SemaphoreType.DMA((NP - 1,)),
        ],
        compiler_params=pltpu.CompilerParams(collective_id=0),
    )(xb, Wdkv.astype(jnp.bfloat16), col_perm(Wuk), col_perm(Wuv),
      wq_h, wqr_h, Wkr.astype(jnp.bfloat16))

    wo_perm = (Wo.astype(jnp.bfloat16)
               .reshape(NP, NP, DH, D)
               .transpose(1, 0, 2, 3)
               .reshape(NP, BW, D))

    out = pl.pallas_call(
        _attn_ag_body,
        out_shape=jax.ShapeDtypeStruct((T, D), jnp.float32),
        in_specs=[pl.BlockSpec(memory_space=pltpu.VMEM)] * 5,
        out_specs=pl.BlockSpec(memory_space=pltpu.VMEM),
        scratch_shapes=[
            pltpu.VMEM((T, DH), jnp.bfloat16),
            pltpu.VMEM((TH, BW), jnp.bfloat16),
            pltpu.VMEM((TH, BW), jnp.bfloat16),
            pltpu.VMEM((NP - 1, TH, DH), jnp.bfloat16),
            pltpu.VMEM((NP - 1, TH, DH), jnp.bfloat16),
            pltpu.VMEM((NP - 1, TH, BW), jnp.bfloat16),
            pltpu.VMEM((NP - 1, TH, BW), jnp.bfloat16),
            pltpu.SemaphoreType.DMA((NP - 1,)),
            pltpu.SemaphoreType.DMA((NP - 1,)),
            pltpu.SemaphoreType.DMA((NP - 1,)),
            pltpu.SemaphoreType.DMA((NP - 1,)),
            pltpu.SemaphoreType.DMA((NP - 1,)),
            pltpu.SemaphoreType.DMA((NP - 1,)),
            pltpu.SemaphoreType.DMA((NP - 1,)),
            pltpu.SemaphoreType.DMA((NP - 1,)),
        ],
        compiler_params=pltpu.CompilerParams(collective_id=1),
    )(kv, q, qr, kr, wo_perm)
    return out.reshape(B, S, D)
